# Initial kernel scaffold; baseline (speedup 1.0000x reference)
import jax
import jax.numpy as jnp
from jax.experimental import pallas as pl


def kernel(pts, W1, g1, b1, W2, g2, b2, W3, g3, b3, W4, g4, b4, W5, g5, b5):
    raise NotImplementedError("write your pallas kernel here")


# speedup vs baseline: 13.7004x; 13.7004x over previous
import functools

import jax
import jax.numpy as jnp
from jax import lax
from jax.experimental import pallas as pl
from jax.experimental.pallas import tpu as pltpu
from jax.experimental.pallas import tpu_sc as plsc

KN = 20
NPTS = 4096
NB = 2

_ROWS = 256


def _knn_body(rows_ref, cols_ref, out_ref, dist_ref):
    rows = rows_ref[0]
    cols = cols_ref[0]
    ra = rows[:, 0:1]
    rb = rows[:, 1:2]
    ca = cols[0:1, :]
    cb = cols[1:2, :]
    xx_r = ra * ra + rb * rb
    xx_c = ca * ca + cb * cb
    rab = ra.astype(jnp.bfloat16).astype(jnp.float32)
    rbb = rb.astype(jnp.bfloat16).astype(jnp.float32)
    cab = ca.astype(jnp.bfloat16).astype(jnp.float32)
    cbb = cb.astype(jnp.bfloat16).astype(jnp.float32)
    inner = -2.0 * (rab * cab + rbb * cbb)
    dist_ref[...] = -xx_r - inner - xx_c
    iota = lax.broadcasted_iota(jnp.int32, (_ROWS, NPTS), 1)
    for j in range(KN):
        d = dist_ref[...]
        m = jnp.max(d, axis=1, keepdims=True)
        cand = jnp.where(d >= m, iota, NPTS)
        arg = jnp.min(cand, axis=1, keepdims=True)
        out_ref[0, :, j:j + 1] = arg
        dist_ref[...] = jnp.where(iota == arg, -jnp.inf, d)


def _knn(pts, pts_rows):
    grid = (NB, NPTS // _ROWS)
    return pl.pallas_call(
        _knn_body,
        grid=grid,
        in_specs=[
            pl.BlockSpec((1, _ROWS, 2), lambda b, j: (b, j, 0)),
            pl.BlockSpec((1, 2, NPTS), lambda b, j: (b, 0, 0)),
        ],
        out_specs=pl.BlockSpec((1, _ROWS, KN), lambda b, j: (b, j, 0)),
        out_shape=jax.ShapeDtypeStruct((NB, NPTS, KN), jnp.int32),
        scratch_shapes=[pltpu.VMEM((_ROWS, NPTS), jnp.float32)],
    )(pts_rows, pts)


_NWORK = 32


def _sc_gather(table, idx, chunk):
    M = idx.shape[0]
    C = table.shape[1]
    per_w = M // _NWORK
    nchunks = per_w // chunk
    mesh = plsc.VectorSubcoreMesh(core_axis_name="c", subcore_axis_name="s")

    @functools.partial(
        pl.kernel,
        mesh=mesh,
        out_type=jax.ShapeDtypeStruct((M, C), jnp.float32),
        scratch_types=[
            pltpu.VMEM((chunk,), jnp.int32),
            pltpu.VMEM((chunk, C), jnp.float32),
            pltpu.SemaphoreType.DMA,
        ],
    )
    def gather_k(table_hbm, idx_hbm, out_hbm, idx_v, rows_v, sem):
        wid = lax.axis_index("s") * 2 + lax.axis_index("c")
        base0 = wid * per_w

        def body(g, carry):
            base = base0 + g * chunk
            pltpu.sync_copy(idx_hbm.at[pl.ds(base, chunk)], idx_v)
            pltpu.async_copy(table_hbm.at[idx_v], rows_v, sem).wait()
            pltpu.sync_copy(rows_v, out_hbm.at[pl.ds(base, chunk)])
            return carry

        lax.fori_loop(0, nchunks, body, 0)

    return gather_k(table, idx)


_RB = 256


def _edge_body(apply_act, cin, cout,
               g_ref, x_ref, w_ref, sc_ref, sh_ref,
               ymax_ref, s_ref, ss_ref):
    def act(v):
        if not apply_act:
            return v
        t = v * sc_ref[...] + sh_ref[...]
        return jnp.where(t > 0, t, 0.2 * t)

    dn = (((1,), (1,)), ((), ()))
    xc = act(x_ref[...])
    acc = jnp.full((_RB, cout), -jnp.inf, jnp.float32)
    ssum = jnp.zeros((1, cout), jnp.float32)
    ssq = jnp.zeros((1, cout), jnp.float32)
    for k in range(KN):
        gk = act(g_ref[k])
        ek = jnp.concatenate([gk - xc, xc], axis=1)
        yk = lax.dot_general(ek, w_ref[...], dn,
                             preferred_element_type=jnp.float32)
        acc = jnp.maximum(acc, yk)
        ssum = ssum + jnp.sum(yk, axis=0, keepdims=True)
        ssq = ssq + jnp.sum(yk * yk, axis=0, keepdims=True)
    ymax_ref[...] = acc

    @pl.when(pl.program_id(0) == 0)
    def _():
        s_ref[...] = jnp.zeros((8, cout), jnp.float32)
        ss_ref[...] = jnp.zeros((8, cout), jnp.float32)

    s_ref[...] += jnp.broadcast_to(ssum, (8, cout))
    ss_ref[...] += jnp.broadcast_to(ssq, (8, cout))


def _edgeconv(gathered, x, w2, sc, sh, cin, cout, apply_act):
    R = x.shape[0]
    grid = (R // _RB,)
    body = functools.partial(_edge_body, apply_act, cin, cout)
    ymax, s, ss = pl.pallas_call(
        body,
        grid=grid,
        in_specs=[
            pl.BlockSpec((KN, _RB, cin), lambda j: (0, j, 0)),
            pl.BlockSpec((_RB, cin), lambda j: (j, 0)),
            pl.BlockSpec((cout, 2 * cin), lambda j: (0, 0)),
            pl.BlockSpec((1, cin), lambda j: (0, 0)),
            pl.BlockSpec((1, cin), lambda j: (0, 0)),
        ],
        out_specs=[
            pl.BlockSpec((_RB, cout), lambda j: (j, 0)),
            pl.BlockSpec((8, cout), lambda j: (0, 0)),
            pl.BlockSpec((8, cout), lambda j: (0, 0)),
        ],
        out_shape=[
            jax.ShapeDtypeStruct((R, cout), jnp.float32),
            jax.ShapeDtypeStruct((8, cout), jnp.float32),
            jax.ShapeDtypeStruct((8, cout), jnp.float32),
        ],
    )(gathered, x, w2, sc, sh)
    return ymax, s[0], ss[0]


def _bn_affine(s, ss, cnt, g, b):
    m = s / cnt
    v = ss / cnt - m * m
    scale = g / jnp.sqrt(v + 1e-5)
    shift = b - m * scale
    return scale, shift


_RB5 = 512
_SIZES = (128, 128, 128, 256)
_OFFS = (0, 128, 256, 384)
_W5C = 640


def _final_body(w_ref, x1_ref, x2_ref, x3_ref, x4_ref,
                a1_ref, a2_ref, a3_ref, a4_ref,
                y_ref, s_ref, ss_ref):
    dn = (((1,), (1,)), ((), ()))
    y = jnp.zeros((1024, _RB5), jnp.float32)
    for x_ref, a_ref, off, cw in zip((x1_ref, x2_ref, x3_ref, x4_ref),
                                     (a1_ref, a2_ref, a3_ref, a4_ref),
                                     _OFFS, _SIZES):
        t = x_ref[...] * a_ref[0:1, :] + a_ref[1:2, :]
        xi = jnp.where(t > 0, t, 0.2 * t)
        wi = w_ref[:, off:off + cw]
        y = y + lax.dot_general(wi, xi, dn, preferred_element_type=jnp.float32)
    y_ref[0] = y

    first = jnp.logical_and(pl.program_id(0) == 0, pl.program_id(1) == 0)

    @pl.when(first)
    def _():
        s_ref[...] = jnp.zeros((1024, 8), jnp.float32)
        ss_ref[...] = jnp.zeros((1024, 8), jnp.float32)

    s_ref[...] += jnp.broadcast_to(jnp.sum(y, axis=1, keepdims=True), (1024, 8))
    ss_ref[...] += jnp.broadcast_to(jnp.sum(y * y, axis=1, keepdims=True), (1024, 8))


def _final(w5, xs, affs):
    nj = NPTS // _RB5
    grid = (NB, nj)
    in_specs = [pl.BlockSpec((1024, _W5C), lambda b, j: (0, 0))]
    for cw in _SIZES:
        in_specs.append(
            pl.BlockSpec((_RB5, cw), lambda b, j: (b * nj + j, 0)))
    for cw in _SIZES:
        in_specs.append(pl.BlockSpec((2, cw), lambda b, j: (0, 0)))
    y, s, ss = pl.pallas_call(
        _final_body,
        grid=grid,
        in_specs=in_specs,
        out_specs=[
            pl.BlockSpec((1, 1024, _RB5), lambda b, j: (b, 0, j)),
            pl.BlockSpec((1024, 8), lambda b, j: (0, 0)),
            pl.BlockSpec((1024, 8), lambda b, j: (0, 0)),
        ],
        out_shape=[
            jax.ShapeDtypeStruct((NB, 1024, NPTS), jnp.float32),
            jax.ShapeDtypeStruct((1024, 8), jnp.float32),
            jax.ShapeDtypeStruct((1024, 8), jnp.float32),
        ],
    )(w5, *xs, *affs)
    return y, s[:, 0], ss[:, 0]


def _apply_body(y_ref, a_ref, o_ref):
    t = y_ref[0] * a_ref[:, 0:1] + a_ref[:, 1:2]
    o_ref[0] = jnp.where(t > 0, t, 0.2 * t)


def _apply(y, a):
    nj = NPTS // _RB5
    return pl.pallas_call(
        _apply_body,
        grid=(NB, nj),
        in_specs=[
            pl.BlockSpec((1, 1024, _RB5), lambda b, j: (b, 0, j)),
            pl.BlockSpec((1024, 2), lambda b, j: (0, 0)),
        ],
        out_specs=pl.BlockSpec((1, 1024, _RB5), lambda b, j: (b, 0, j)),
        out_shape=jax.ShapeDtypeStruct((NB, 1024, NPTS), jnp.float32),
    )(y, a)


def _pad2(a, rows, cols):
    return jnp.pad(a, ((0, rows - a.shape[0]), (0, cols - a.shape[1])))


def kernel(pts, W1, g1, b1, W2, g2, b2, W3, g3, b3, W4, g4, b4, W5, g5, b5):
    B, D, N = pts.shape
    R = B * N
    cnt_e = float(R * KN)
    CH = 128

    pts_rows = jnp.transpose(pts, (0, 2, 1))
    idx = _knn(pts, pts_rows)
    idx_g = (jnp.transpose(idx, (2, 0, 1))
             + (jnp.arange(B, dtype=jnp.int32) * N)[None, :, None])
    idx_flat = idx_g.reshape(-1)

    x0 = jnp.pad(pts_rows.reshape(R, D), ((0, 0), (0, 128 - D)))
    g0 = _sc_gather(x0, idx_flat, CH).reshape(KN, R, 128)
    w1p = jnp.concatenate(
        [_pad2(W1[:, :D], 128, 128), _pad2(W1[:, D:], 128, 128)], axis=1)
    one128 = jnp.ones((1, 128), jnp.float32)
    zero128 = jnp.zeros((1, 128), jnp.float32)
    y1, s1, ss1 = _edgeconv(g0, x0, w1p, one128, zero128, 128, 128, False)
    sc1, sh1 = _bn_affine(s1, ss1, cnt_e, jnp.pad(g1, (0, 64)),
                          jnp.pad(b1, (0, 64)))

    gt = _sc_gather(y1, idx_flat, CH).reshape(KN, R, 128)
    w2p = jnp.concatenate(
        [_pad2(W2[:, :64], 128, 128), _pad2(W2[:, 64:], 128, 128)], axis=1)
    y2, s2, ss2 = _edgeconv(gt, y1, w2p, sc1.reshape(1, -1), sh1.reshape(1, -1),
                            128, 128, True)
    sc2, sh2 = _bn_affine(s2, ss2, cnt_e, jnp.pad(g2, (0, 64)),
                          jnp.pad(b2, (0, 64)))

    gt = _sc_gather(y2, idx_flat, CH).reshape(KN, R, 128)
    w3p = jnp.concatenate(
        [_pad2(W3[:, :64], 128, 128), _pad2(W3[:, 64:], 128, 128)], axis=1)
    y3, s3, ss3 = _edgeconv(gt, y2, w3p, sc2.reshape(1, -1), sh2.reshape(1, -1),
                            128, 128, True)
    sc3, sh3 = _bn_affine(s3, ss3, cnt_e, g3, b3)

    gt = _sc_gather(y3, idx_flat, CH).reshape(KN, R, 128)
    y4, s4, ss4 = _edgeconv(gt, y3, W4, sc3.reshape(1, -1), sh3.reshape(1, -1),
                            128, 256, True)
    sc4, sh4 = _bn_affine(s4, ss4, cnt_e, g4, b4)

    w5p = jnp.concatenate(
        [_pad2(W5[:, 0:64], 1024, 128), _pad2(W5[:, 64:128], 1024, 128),
         W5[:, 128:256], W5[:, 256:512]], axis=1)
    affs = [jnp.concatenate([sc.reshape(1, -1), sh.reshape(1, -1)], axis=0)
            for sc, sh in ((sc1, sh1), (sc2, sh2), (sc3, sh3), (sc4, sh4))]
    y5, s5, ss5 = _final(w5p, (y1, y2, y3, y4), affs)
    sc5, sh5 = _bn_affine(s5, ss5, float(R), g5, b5)
    a5 = jnp.stack([sc5, sh5], axis=1)
    return _apply(y5, a5)


# speedup vs baseline: 15.4003x; 1.1241x over previous
import functools

import jax
import jax.numpy as jnp
from jax import lax
from jax.experimental import pallas as pl
from jax.experimental.pallas import tpu as pltpu
from jax.experimental.pallas import tpu_sc as plsc

KN = 20
NPTS = 4096
NB = 2

_ROWS = 256


def _knn_body(rows_ref, cols_ref, out_ref, dist_ref):
    rows = rows_ref[0]
    cols = cols_ref[0]
    ra = rows[:, 0:1]
    rb = rows[:, 1:2]
    ca = cols[0:1, :]
    cb = cols[1:2, :]
    xx_r = ra * ra + rb * rb
    xx_c = ca * ca + cb * cb
    rab = ra.astype(jnp.bfloat16).astype(jnp.float32)
    rbb = rb.astype(jnp.bfloat16).astype(jnp.float32)
    cab = ca.astype(jnp.bfloat16).astype(jnp.float32)
    cbb = cb.astype(jnp.bfloat16).astype(jnp.float32)
    inner = -2.0 * (rab * cab + rbb * cbb)
    dist_ref[...] = -xx_r - inner - xx_c
    iota = lax.broadcasted_iota(jnp.int32, (_ROWS, NPTS), 1)
    for j in range(KN):
        d = dist_ref[...]
        m = jnp.max(d, axis=1, keepdims=True)
        cand = jnp.where(d >= m, iota, NPTS)
        arg = jnp.min(cand, axis=1, keepdims=True)
        out_ref[0, :, j:j + 1] = arg
        dist_ref[...] = jnp.where(iota == arg, -jnp.inf, d)


def _knn(pts, pts_rows):
    grid = (NB, NPTS // _ROWS)
    return pl.pallas_call(
        _knn_body,
        grid=grid,
        in_specs=[
            pl.BlockSpec((1, _ROWS, 2), lambda b, j: (b, j, 0)),
            pl.BlockSpec((1, 2, NPTS), lambda b, j: (b, 0, 0)),
        ],
        out_specs=pl.BlockSpec((1, _ROWS, KN), lambda b, j: (b, j, 0)),
        out_shape=jax.ShapeDtypeStruct((NB, NPTS, KN), jnp.int32),
        scratch_shapes=[pltpu.VMEM((_ROWS, NPTS), jnp.float32)],
    )(pts_rows, pts)


_NWORK = 32


def _sc_gather(table, idx, chunk):
    M = idx.shape[0]
    C = table.shape[1]
    per_w = M // _NWORK
    nchunks = per_w // chunk
    mesh = plsc.VectorSubcoreMesh(core_axis_name="c", subcore_axis_name="s")

    @functools.partial(
        pl.kernel,
        mesh=mesh,
        out_type=jax.ShapeDtypeStruct((M, C), jnp.float32),
        scratch_types=[
            pltpu.VMEM((chunk,), jnp.int32),
            pltpu.VMEM((chunk, C), jnp.float32),
            pltpu.SemaphoreType.DMA,
        ],
    )
    def gather_k(table_hbm, idx_hbm, out_hbm, idx_v, rows_v, sem):
        wid = lax.axis_index("s") * 2 + lax.axis_index("c")
        base0 = wid * per_w

        def body(g, carry):
            base = base0 + g * chunk
            pltpu.sync_copy(idx_hbm.at[pl.ds(base, chunk)], idx_v)
            pltpu.async_copy(table_hbm.at[idx_v], rows_v, sem).wait()
            pltpu.sync_copy(rows_v, out_hbm.at[pl.ds(base, chunk)])
            return carry

        lax.fori_loop(0, nchunks, body, 0)

    return gather_k(table, idx)


_RB = 256


def _edge_body(apply_act, cin, cout,
               g_ref, x_ref, w_ref, sc_ref, sh_ref,
               ymax_ref, s_ref, ss_ref):
    def act(v):
        if not apply_act:
            return v
        t = v * sc_ref[...] + sh_ref[...]
        return jnp.where(t > 0, t, 0.2 * t)

    dn = (((1,), (1,)), ((), ()))
    xc = act(x_ref[...])
    acc = jnp.full((_RB, cout), -jnp.inf, jnp.float32)
    ssum = jnp.zeros((1, cout), jnp.float32)
    ssq = jnp.zeros((1, cout), jnp.float32)
    for k in range(KN):
        gk = act(g_ref[k])
        ek = jnp.concatenate([gk - xc, xc], axis=1)
        yk = lax.dot_general(ek, w_ref[...], dn,
                             preferred_element_type=jnp.float32)
        acc = jnp.maximum(acc, yk)
        ssum = ssum + jnp.sum(yk, axis=0, keepdims=True)
        ssq = ssq + jnp.sum(yk * yk, axis=0, keepdims=True)
    ymax_ref[...] = acc

    @pl.when(pl.program_id(0) == 0)
    def _():
        s_ref[...] = jnp.zeros((8, cout), jnp.float32)
        ss_ref[...] = jnp.zeros((8, cout), jnp.float32)

    s_ref[...] += jnp.broadcast_to(ssum, (8, cout))
    ss_ref[...] += jnp.broadcast_to(ssq, (8, cout))


def _edgeconv(gathered, x, w2, sc, sh, cin, cout, apply_act):
    R = x.shape[0]
    grid = (R // _RB,)
    body = functools.partial(_edge_body, apply_act, cin, cout)
    ymax, s, ss = pl.pallas_call(
        body,
        grid=grid,
        in_specs=[
            pl.BlockSpec((KN, _RB, cin), lambda j: (0, j, 0)),
            pl.BlockSpec((_RB, cin), lambda j: (j, 0)),
            pl.BlockSpec((cout, 2 * cin), lambda j: (0, 0)),
            pl.BlockSpec((1, cin), lambda j: (0, 0)),
            pl.BlockSpec((1, cin), lambda j: (0, 0)),
        ],
        out_specs=[
            pl.BlockSpec((_RB, cout), lambda j: (j, 0)),
            pl.BlockSpec((8, cout), lambda j: (0, 0)),
            pl.BlockSpec((8, cout), lambda j: (0, 0)),
        ],
        out_shape=[
            jax.ShapeDtypeStruct((R, cout), jnp.float32),
            jax.ShapeDtypeStruct((8, cout), jnp.float32),
            jax.ShapeDtypeStruct((8, cout), jnp.float32),
        ],
    )(gathered, x, w2, sc, sh)
    return ymax, s[0], ss[0]


def _bn_affine(s, ss, cnt, g, b):
    m = s / cnt
    v = ss / cnt - m * m
    scale = g / jnp.sqrt(v + 1e-5)
    shift = b - m * scale
    return scale, shift


_RB5 = 512
_SIZES = (128, 128, 128, 256)
_OFFS = (0, 128, 256, 384)
_W5C = 640


def _final_body(w_ref, x1_ref, x2_ref, x3_ref, x4_ref,
                a1_ref, a2_ref, a3_ref, a4_ref,
                y_ref, s_ref, ss_ref):
    dn = (((1,), (1,)), ((), ()))
    y = jnp.zeros((1024, _RB5), jnp.float32)
    for x_ref, a_ref, off, cw in zip((x1_ref, x2_ref, x3_ref, x4_ref),
                                     (a1_ref, a2_ref, a3_ref, a4_ref),
                                     _OFFS, _SIZES):
        t = x_ref[...] * a_ref[0:1, :] + a_ref[1:2, :]
        xi = jnp.where(t > 0, t, 0.2 * t)
        wi = w_ref[:, off:off + cw]
        y = y + lax.dot_general(wi, xi, dn, preferred_element_type=jnp.float32)
    y_ref[0] = y

    first = jnp.logical_and(pl.program_id(0) == 0, pl.program_id(1) == 0)

    @pl.when(first)
    def _():
        s_ref[...] = jnp.zeros((1024, 8), jnp.float32)
        ss_ref[...] = jnp.zeros((1024, 8), jnp.float32)

    s_ref[...] += jnp.broadcast_to(jnp.sum(y, axis=1, keepdims=True), (1024, 8))
    ss_ref[...] += jnp.broadcast_to(jnp.sum(y * y, axis=1, keepdims=True), (1024, 8))


def _final(w5, xs, affs):
    nj = NPTS // _RB5
    grid = (NB, nj)
    in_specs = [pl.BlockSpec((1024, _W5C), lambda b, j: (0, 0))]
    for cw in _SIZES:
        in_specs.append(
            pl.BlockSpec((_RB5, cw), lambda b, j: (b * nj + j, 0)))
    for cw in _SIZES:
        in_specs.append(pl.BlockSpec((2, cw), lambda b, j: (0, 0)))
    y, s, ss = pl.pallas_call(
        _final_body,
        grid=grid,
        in_specs=in_specs,
        out_specs=[
            pl.BlockSpec((1, 1024, _RB5), lambda b, j: (b, 0, j)),
            pl.BlockSpec((1024, 8), lambda b, j: (0, 0)),
            pl.BlockSpec((1024, 8), lambda b, j: (0, 0)),
        ],
        out_shape=[
            jax.ShapeDtypeStruct((NB, 1024, NPTS), jnp.float32),
            jax.ShapeDtypeStruct((1024, 8), jnp.float32),
            jax.ShapeDtypeStruct((1024, 8), jnp.float32),
        ],
    )(w5, *xs, *affs)
    return y, s[:, 0], ss[:, 0]


def _apply_body(y_ref, a_ref, o_ref):
    t = y_ref[0] * a_ref[:, 0:1] + a_ref[:, 1:2]
    o_ref[0] = jnp.where(t > 0, t, 0.2 * t)


def _apply(y, a):
    nj = NPTS // _RB5
    return pl.pallas_call(
        _apply_body,
        grid=(NB, nj),
        in_specs=[
            pl.BlockSpec((1, 1024, _RB5), lambda b, j: (b, 0, j)),
            pl.BlockSpec((1024, 2), lambda b, j: (0, 0)),
        ],
        out_specs=pl.BlockSpec((1, 1024, _RB5), lambda b, j: (b, 0, j)),
        out_shape=jax.ShapeDtypeStruct((NB, 1024, NPTS), jnp.float32),
    )(y, a)


def _pad2(a, rows, cols):
    return jnp.pad(a, ((0, rows - a.shape[0]), (0, cols - a.shape[1])))


def kernel(pts, W1, g1, b1, W2, g2, b2, W3, g3, b3, W4, g4, b4, W5, g5, b5):
    B, D, N = pts.shape
    R = B * N
    cnt_e = float(R * KN)
    CH = 512

    pts_rows = jnp.transpose(pts, (0, 2, 1))
    idx = _knn(pts, pts_rows)
    idx_g = (jnp.transpose(idx, (2, 0, 1))
             + (jnp.arange(B, dtype=jnp.int32) * N)[None, :, None])
    idx_flat = idx_g.reshape(-1)

    x0 = jnp.pad(pts_rows.reshape(R, D), ((0, 0), (0, 128 - D)))
    g0 = _sc_gather(x0, idx_flat, CH).reshape(KN, R, 128)
    w1p = jnp.concatenate(
        [_pad2(W1[:, :D], 128, 128), _pad2(W1[:, D:], 128, 128)], axis=1)
    one128 = jnp.ones((1, 128), jnp.float32)
    zero128 = jnp.zeros((1, 128), jnp.float32)
    y1, s1, ss1 = _edgeconv(g0, x0, w1p, one128, zero128, 128, 128, False)
    sc1, sh1 = _bn_affine(s1, ss1, cnt_e, jnp.pad(g1, (0, 64)),
                          jnp.pad(b1, (0, 64)))

    gt = _sc_gather(y1, idx_flat, CH).reshape(KN, R, 128)
    w2p = jnp.concatenate(
        [_pad2(W2[:, :64], 128, 128), _pad2(W2[:, 64:], 128, 128)], axis=1)
    y2, s2, ss2 = _edgeconv(gt, y1, w2p, sc1.reshape(1, -1), sh1.reshape(1, -1),
                            128, 128, True)
    sc2, sh2 = _bn_affine(s2, ss2, cnt_e, jnp.pad(g2, (0, 64)),
                          jnp.pad(b2, (0, 64)))

    gt = _sc_gather(y2, idx_flat, CH).reshape(KN, R, 128)
    w3p = jnp.concatenate(
        [_pad2(W3[:, :64], 128, 128), _pad2(W3[:, 64:], 128, 128)], axis=1)
    y3, s3, ss3 = _edgeconv(gt, y2, w3p, sc2.reshape(1, -1), sh2.reshape(1, -1),
                            128, 128, True)
    sc3, sh3 = _bn_affine(s3, ss3, cnt_e, g3, b3)

    gt = _sc_gather(y3, idx_flat, CH).reshape(KN, R, 128)
    y4, s4, ss4 = _edgeconv(gt, y3, W4, sc3.reshape(1, -1), sh3.reshape(1, -1),
                            128, 256, True)
    sc4, sh4 = _bn_affine(s4, ss4, cnt_e, g4, b4)

    w5p = jnp.concatenate(
        [_pad2(W5[:, 0:64], 1024, 128), _pad2(W5[:, 64:128], 1024, 128),
         W5[:, 128:256], W5[:, 256:512]], axis=1)
    affs = [jnp.concatenate([sc.reshape(1, -1), sh.reshape(1, -1)], axis=0)
            for sc, sh in ((sc1, sh1), (sc2, sh2), (sc3, sh3), (sc4, sh4))]
    y5, s5, ss5 = _final(w5p, (y1, y2, y3, y4), affs)
    sc5, sh5 = _bn_affine(s5, ss5, float(R), g5, b5)
    a5 = jnp.stack([sc5, sh5], axis=1)
    return _apply(y5, a5)
